# baseline (speedup 1.0000x reference)
import functools

import jax
import jax.numpy as jnp
from jax import lax
from jax.experimental import pallas as pl
from jax.experimental.pallas import tpu as pltpu
from jax.experimental.pallas import tpu_sc as plsc

EMBED = 128
BLOCK = 128
NC, NS = 2, 16
NW = NC * NS
NBUF = 7
LEAD = 4
LAG = 3


@jax.jit
def _sc_gather(idx_t, table):
    seq, batch = idx_t.shape
    n_outer = seq // NBUF
    mesh = plsc.VectorSubcoreMesh(core_axis_name="c", subcore_axis_name="s")

    @functools.partial(
        pl.kernel,
        mesh=mesh,
        out_type=jax.ShapeDtypeStruct((seq, batch, EMBED), jnp.float32),
        scratch_types=[
            pltpu.VMEM((seq, BLOCK), jnp.int32),
            pltpu.VMEM((NBUF, BLOCK, EMBED), jnp.float32),
        ]
        + [pltpu.SemaphoreType.DMA] * (2 * NBUF),
    )
    def k(idx_hbm, table_hbm, out_hbm, idx_v, rows_v, *sems):
        gsem, ssem = sems[:NBUF], sems[NBUF:]
        wid = lax.axis_index("s") * NC + lax.axis_index("c")
        col0 = wid * BLOCK
        pltpu.sync_copy(idx_hbm.at[:, pl.ds(col0, BLOCK)], idx_v)

        def gather(slot, s):
            return pltpu.make_async_copy(
                table_hbm.at[pl.ds(s * BLOCK, BLOCK)], rows_v.at[slot], gsem[slot]
            )

        def store(slot, s):
            return pltpu.make_async_copy(
                rows_v.at[slot],
                out_hbm.at[s].at[pl.ds(col0, BLOCK)],
                ssem[slot],
            )

        for slot in range(LEAD):
            gather(slot, slot).start()

        def outer(t, _):
            for slot in range(NBUF):
                j = t * NBUF + slot
                gather(slot, j).wait()
                store(slot, j).start()

                if slot >= LAG:
                    store(slot - LAG, j - LAG).wait()
                else:

                    @pl.when(t > 0)
                    def _():
                        store((slot - LAG) % NBUF, j - LAG).wait()

                if slot + LEAD < NBUF:
                    gather(slot + LEAD, j + LEAD).start()
                elif slot + LEAD - NBUF <= (seq - 1) % NBUF:
                    gather((slot + LEAD) % NBUF, j + LEAD).start()
                else:

                    @pl.when(t < n_outer - 1)
                    def _():
                        gather((slot + LEAD) % NBUF, j + LEAD).start()

            return 0

        lax.fori_loop(0, n_outer, outer, 0)

        for j in range(n_outer * NBUF, seq):
            slot = j % NBUF
            gather(slot, j).wait()
            store(slot, j).start()
        for j in range(n_outer * NBUF - LAG, seq):
            store(j % NBUF, j).wait()

    return k(idx_t, table)


def kernel(inputs, table):
    out = _sc_gather(inputs.T, table)
    return out.transpose(1, 0, 2)


# speedup vs baseline: 1.3324x; 1.3324x over previous
import functools

import jax
import jax.numpy as jnp
from jax import lax
from jax.experimental import pallas as pl
from jax.experimental.pallas import tpu as pltpu
from jax.experimental.pallas import tpu_sc as plsc

EMBED = 128
BLOCK = 128
NC, NS = 2, 16
NW = NC * NS
NBUF = 5
LEAD = 3
LAG = 2


@jax.jit
def _sc_gather(idx_t, table):
    seq, batch = idx_t.shape
    mesh = plsc.VectorSubcoreMesh(core_axis_name="c", subcore_axis_name="s")

    @functools.partial(
        pl.kernel,
        mesh=mesh,
        out_type=jax.ShapeDtypeStruct((seq, batch, EMBED), jnp.float32),
        scratch_types=[
            pltpu.VMEM((seq, BLOCK), jnp.int32),
            pltpu.VMEM((NBUF, BLOCK, EMBED), jnp.float32),
            pltpu.SemaphoreType.DMA((NBUF,)),
            pltpu.SemaphoreType.DMA((NBUF,)),
        ],
    )
    def k(idx_hbm, table_hbm, out_hbm, idx_v, rows_v, gsem, ssem):
        wid = lax.axis_index("s") * NC + lax.axis_index("c")
        col0 = wid * BLOCK
        pltpu.sync_copy(idx_hbm.at[:, pl.ds(col0, BLOCK)], idx_v)

        def gather(slot, s):
            return pltpu.make_async_copy(
                table_hbm.at[idx_v.at[s]], rows_v.at[slot], gsem.at[slot]
            )

        def store(slot, s):
            return pltpu.make_async_copy(
                rows_v.at[slot],
                out_hbm.at[s].at[pl.ds(col0, BLOCK)],
                ssem.at[slot],
            )

        for j in range(LEAD):
            gather(j, j).start()

        def body(j, _):
            slot = lax.rem(j, NBUF)
            gather(slot, j).wait()
            store(slot, j).start()

            @pl.when(j >= LAG)
            def _():
                store(lax.rem(j - LAG, NBUF), j - LAG).wait()

            @pl.when(j + LEAD < seq)
            def _():
                gather(lax.rem(j + LEAD, NBUF), j + LEAD).start()

            return 0

        lax.fori_loop(0, seq, body, 0)
        for j in range(seq - LAG, seq):
            store(j % NBUF, j).wait()

    return k(idx_t, table)


def kernel(inputs, table):
    out = _sc_gather(inputs.T, table)
    return out.transpose(1, 0, 2)


# speedup vs baseline: 1.3395x; 1.0053x over previous
import functools

import jax
import jax.numpy as jnp
from jax import lax
from jax.experimental import pallas as pl
from jax.experimental.pallas import tpu as pltpu
from jax.experimental.pallas import tpu_sc as plsc

EMBED = 128
BLOCK = 128
NC, NS = 2, 16
NW = NC * NS
NBUF = 7
LEAD = 4
LAG = 3


@jax.jit
def _sc_gather(idx_t, table):
    seq, batch = idx_t.shape
    n_outer = seq // NBUF
    mesh = plsc.VectorSubcoreMesh(core_axis_name="c", subcore_axis_name="s")

    @functools.partial(
        pl.kernel,
        mesh=mesh,
        out_type=jax.ShapeDtypeStruct((seq, batch, EMBED), jnp.float32),
        scratch_types=[
            pltpu.VMEM((seq, BLOCK), jnp.int32),
            pltpu.VMEM((NBUF, BLOCK, EMBED), jnp.float32),
        ]
        + [pltpu.SemaphoreType.DMA] * (2 * NBUF),
    )
    def k(idx_hbm, table_hbm, out_hbm, idx_v, rows_v, *sems):
        gsem, ssem = sems[:NBUF], sems[NBUF:]
        wid = lax.axis_index("s") * NC + lax.axis_index("c")
        col0 = wid * BLOCK
        pltpu.sync_copy(idx_hbm.at[:, pl.ds(col0, BLOCK)], idx_v)

        def gather(slot, s):
            return pltpu.make_async_copy(
                table_hbm.at[idx_v.at[s]], rows_v.at[slot], gsem[slot]
            )

        def store(slot, s):
            return pltpu.make_async_copy(
                rows_v.at[slot],
                out_hbm.at[s].at[pl.ds(col0, BLOCK)],
                ssem[slot],
            )

        for slot in range(LEAD):
            gather(slot, slot).start()

        def outer(t, _):
            for slot in range(NBUF):
                j = t * NBUF + slot
                gather(slot, j).wait()
                store(slot, j).start()

                if slot >= LAG:
                    store(slot - LAG, j - LAG).wait()
                else:

                    @pl.when(t > 0)
                    def _():
                        store((slot - LAG) % NBUF, j - LAG).wait()

                if slot + LEAD < NBUF:
                    gather(slot + LEAD, j + LEAD).start()
                elif slot + LEAD - NBUF <= (seq - 1) % NBUF:
                    gather((slot + LEAD) % NBUF, j + LEAD).start()
                else:

                    @pl.when(t < n_outer - 1)
                    def _():
                        gather((slot + LEAD) % NBUF, j + LEAD).start()

            return 0

        lax.fori_loop(0, n_outer, outer, 0)

        for j in range(n_outer * NBUF, seq):
            slot = j % NBUF
            gather(slot, j).wait()
            store(slot, j).start()
        for j in range(n_outer * NBUF - LAG, seq):
            store(j % NBUF, j).wait()

    return k(idx_t, table)


def kernel(inputs, table):
    out = _sc_gather(inputs.T, table)
    return out.transpose(1, 0, 2)
